# Initial kernel scaffold; baseline (speedup 1.0000x reference)
import functools

import jax
import jax.numpy as jnp
from jax.experimental import pallas as pl
from jax.experimental.pallas import tpu as pltpu

N = 10000
E = 160000
DIN = 256
DHID = 512
DOUT = 256

BLK = 400
NB = N // BLK
DH2 = DHID // 2


def _k1_body(x_ref, w_ref, degp_ref, hlo_ref, hhi_ref, dinv_ref, cnti_ref):
    h = jnp.dot(x_ref[...], w_ref[...], preferred_element_type=jnp.float32)
    deg = degp_ref[0, :] + degp_ref[1, :] + 1.0
    dinv = jax.lax.rsqrt(deg)
    hs = h * dinv[:, None]
    hlo_ref[...] = hs[:, :DH2]
    hhi_ref[...] = hs[:, DH2:]
    dinv_ref[...] = dinv[:, None]
    cnti_ref[...] = (1.0 / jnp.maximum(deg - 1.0, 1.0))[:, None]


def _run_k1(x, W1, degp):
    return pl.pallas_call(
        _k1_body,
        grid=(NB,),
        in_specs=[
            pl.BlockSpec((BLK, DIN), lambda i: (i, 0)),
            pl.BlockSpec((DIN, DHID), lambda i: (0, 0)),
            pl.BlockSpec((2, BLK), lambda i: (0, i)),
        ],
        out_specs=[
            pl.BlockSpec((BLK, DH2), lambda i: (i, 0)),
            pl.BlockSpec((BLK, DH2), lambda i: (i, 0)),
            pl.BlockSpec((BLK, 1), lambda i: (i, 0)),
            pl.BlockSpec((BLK, 1), lambda i: (i, 0)),
        ],
        out_shape=[
            jax.ShapeDtypeStruct((N, DH2), jnp.float32),
            jax.ShapeDtypeStruct((N, DH2), jnp.float32),
            jax.ShapeDtypeStruct((N, 1), jnp.float32),
            jax.ShapeDtypeStruct((N, 1), jnp.float32),
        ],
    )(x, W1, degp)


def _k2_body(s1lo_ref, s1hi_ref, hlo_ref, hhi_ref, dinv_ref, b1_ref,
             g1_ref, be1_ref, h2lo_ref, h2hi_ref, acc_ref):
    i = pl.program_id(0)
    j = pl.program_id(1)
    s1 = jnp.concatenate([s1lo_ref[...], s1hi_ref[...]], axis=1)
    hs = jnp.concatenate([hlo_ref[...], hhi_ref[...]], axis=1)
    pre = dinv_ref[...] * (s1 + hs) + b1_ref[...]

    @pl.when(i == 0)
    def _():
        @pl.when(j == 0)
        def _():
            acc_ref[...] = jnp.zeros_like(acc_ref)
        acc_ref[0, :] += jnp.sum(pre, axis=0)
        acc_ref[1, :] += jnp.sum(pre * pre, axis=0)

    @pl.when(i == 1)
    def _():
        mean = acc_ref[0, :] / N
        var = acc_ref[1, :] / N - mean * mean
        h2 = (pre - mean) * jax.lax.rsqrt(var + 1e-5) * g1_ref[0, :] \
            + be1_ref[0, :]
        h2 = jnp.maximum(h2, 0.0)
        h2lo_ref[...] = h2[:, :DH2]
        h2hi_ref[...] = h2[:, DH2:]


def _run_k2(s1lo, s1hi, hlo, hhi, dinv, b1, g1, be1):
    row = pl.BlockSpec((BLK, DH2), lambda i, j: (j, 0))
    vec = pl.BlockSpec((1, DHID), lambda i, j: (0, 0))
    return pl.pallas_call(
        _k2_body,
        grid=(2, NB),
        in_specs=[row, row, row, row,
                  pl.BlockSpec((BLK, 1), lambda i, j: (j, 0)),
                  vec, vec, vec],
        out_specs=[row, row],
        out_shape=[
            jax.ShapeDtypeStruct((N, DH2), jnp.float32),
            jax.ShapeDtypeStruct((N, DH2), jnp.float32),
        ],
        scratch_shapes=[pltpu.VMEM((2, DHID), jnp.float32)],
    )(s1lo, s1hi, hlo, hhi, dinv, b1, g1, be1)


def _k3_body(s2lo_ref, s2hi_ref, h2lo_ref, h2hi_ref, cnti_ref, wl_ref,
             wr_ref, bl_ref, g2_ref, be2_ref, out_ref, acc_ref):
    i = pl.program_id(0)
    j = pl.program_id(1)
    s2 = jnp.concatenate([s2lo_ref[...], s2hi_ref[...]], axis=1)
    h2 = jnp.concatenate([h2lo_ref[...], h2hi_ref[...]], axis=1)
    agg = s2 * cnti_ref[...]
    pre = (jnp.dot(agg, wl_ref[...], preferred_element_type=jnp.float32)
           + jnp.dot(h2, wr_ref[...], preferred_element_type=jnp.float32)
           + bl_ref[...])

    @pl.when(i == 0)
    def _():
        @pl.when(j == 0)
        def _():
            acc_ref[...] = jnp.zeros_like(acc_ref)
        acc_ref[0, :] += jnp.sum(pre, axis=0)
        acc_ref[1, :] += jnp.sum(pre * pre, axis=0)

    @pl.when(i == 1)
    def _():
        mean = acc_ref[0, :] / N
        var = acc_ref[1, :] / N - mean * mean
        out = (pre - mean) * jax.lax.rsqrt(var + 1e-5) * g2_ref[0, :] \
            + be2_ref[0, :]
        out_ref[...] = jnp.maximum(out, 0.0)


def _run_k3(s2lo, s2hi, h2lo, h2hi, cnti, Wl, Wr, bl, g2, be2):
    row = pl.BlockSpec((BLK, DH2), lambda i, j: (j, 0))
    vec = pl.BlockSpec((1, DOUT), lambda i, j: (0, 0))
    wspec = pl.BlockSpec((DHID, DOUT), lambda i, j: (0, 0))
    return pl.pallas_call(
        _k3_body,
        grid=(2, NB),
        in_specs=[row, row, row, row,
                  pl.BlockSpec((BLK, 1), lambda i, j: (j, 0)),
                  wspec, wspec, vec, vec, vec],
        out_specs=pl.BlockSpec((BLK, DOUT), lambda i, j: (j, 0)),
        out_shape=jax.ShapeDtypeStruct((N, DOUT), jnp.float32),
        scratch_shapes=[pltpu.VMEM((2, DOUT), jnp.float32)],
    )(s2lo, s2hi, h2lo, h2hi, cnti, Wl, Wr, bl, g2, be2)


def _deg_parts(dst):
    deg = jnp.zeros((N,), jnp.float32).at[dst].add(1.0)
    return jnp.stack([deg, jnp.zeros((N,), jnp.float32)])


def _scatter_rows(src, dst, tlo, thi):
    slo = jnp.zeros((N, DH2), jnp.float32).at[dst].add(tlo[src])
    shi = jnp.zeros((N, DH2), jnp.float32).at[dst].add(thi[src])
    return slo, shi


def kernel(x, edge_index, W1, b1, g1, beta1, Wl, bl, Wr, g2, beta2):
    src = edge_index[0].astype(jnp.int32)
    dst = edge_index[1].astype(jnp.int32)

    degp = _deg_parts(dst)
    hlo, hhi, dinv, cnti = _run_k1(x, W1, degp)
    s1lo, s1hi = _scatter_rows(src, dst, hlo, hhi)
    h2lo, h2hi = _run_k2(s1lo, s1hi, hlo, hhi, dinv,
                         b1.reshape(1, DHID), g1.reshape(1, DHID),
                         beta1.reshape(1, DHID))
    s2lo, s2hi = _scatter_rows(src, dst, h2lo, h2hi)
    out = _run_k3(s2lo, s2hi, h2lo, h2hi, cnti, Wl, Wr,
                  bl.reshape(1, DOUT), g2.reshape(1, DOUT),
                  beta2.reshape(1, DOUT))
    return out


# speedup vs baseline: 1.9461x; 1.9461x over previous
import functools

import jax
import jax.numpy as jnp
from jax.experimental import pallas as pl
from jax.experimental.pallas import tpu as pltpu

N = 10000
E = 160000
DIN = 256
DHID = 512
DOUT = 256

BLK = 400
NB = N // BLK
DH2 = DHID // 2


def _k1_body(x_ref, w_ref, degp_ref, hlo_ref, hhi_ref, dinv_ref, cnti_ref):
    h = jnp.dot(x_ref[...], w_ref[...], preferred_element_type=jnp.float32)
    deg = degp_ref[:, 0] + degp_ref[:, 1] + 1.0
    dinv = jax.lax.rsqrt(deg)
    hs = h * dinv[:, None]
    hlo_ref[...] = hs[:, :DH2]
    hhi_ref[...] = hs[:, DH2:]
    dinv_ref[...] = dinv[:, None]
    cnti_ref[...] = (1.0 / jnp.maximum(deg - 1.0, 1.0))[:, None]


def _run_k1(x, W1, degp):
    return pl.pallas_call(
        _k1_body,
        grid=(NB,),
        in_specs=[
            pl.BlockSpec((BLK, DIN), lambda i: (i, 0)),
            pl.BlockSpec((DIN, DHID), lambda i: (0, 0)),
            pl.BlockSpec((BLK, 2), lambda i: (i, 0)),
        ],
        out_specs=[
            pl.BlockSpec((BLK, DH2), lambda i: (i, 0)),
            pl.BlockSpec((BLK, DH2), lambda i: (i, 0)),
            pl.BlockSpec((BLK, 1), lambda i: (i, 0)),
            pl.BlockSpec((BLK, 1), lambda i: (i, 0)),
        ],
        out_shape=[
            jax.ShapeDtypeStruct((N, DH2), jnp.float32),
            jax.ShapeDtypeStruct((N, DH2), jnp.float32),
            jax.ShapeDtypeStruct((N, 1), jnp.float32),
            jax.ShapeDtypeStruct((N, 1), jnp.float32),
        ],
    )(x, W1, degp)


def _k2a_body(s1lo_ref, s1hi_ref, hlo_ref, hhi_ref, dinv_ref, b1_ref,
              pre_ref, acc_ref):
    j = pl.program_id(0)
    s1 = jnp.concatenate([s1lo_ref[...], s1hi_ref[...]], axis=1)
    hs = jnp.concatenate([hlo_ref[...], hhi_ref[...]], axis=1)
    pre = dinv_ref[...] * (s1 + hs) + b1_ref[...]
    pre_ref[...] = pre

    @pl.when(j == 0)
    def _():
        acc_ref[...] = jnp.zeros_like(acc_ref)
    acc_ref[0, :] += jnp.sum(pre, axis=0)
    acc_ref[1, :] += jnp.sum(pre * pre, axis=0)


def _bn_body(pre_ref, acc_ref, g_ref, be_ref, lo_ref, hi_ref):
    d = pre_ref.shape[1]
    mean = acc_ref[0, :] / N
    var = acc_ref[1, :] / N - mean * mean
    h = (pre_ref[...] - mean) * jax.lax.rsqrt(var + 1e-5) * g_ref[0, :] \
        + be_ref[0, :]
    h = jnp.maximum(h, 0.0)
    lo_ref[...] = h[:, :d // 2]
    hi_ref[...] = h[:, d // 2:]


def _run_bn(pre, acc, g, be):
    d = pre.shape[1]
    vec = pl.BlockSpec((1, d), lambda j: (0, 0))
    half = pl.BlockSpec((BLK, d // 2), lambda j: (j, 0))
    return pl.pallas_call(
        _bn_body,
        grid=(NB,),
        in_specs=[pl.BlockSpec((BLK, d), lambda j: (j, 0)),
                  pl.BlockSpec((2, d), lambda j: (0, 0)),
                  vec, vec],
        out_specs=[half, half],
        out_shape=[
            jax.ShapeDtypeStruct((N, d // 2), jnp.float32),
            jax.ShapeDtypeStruct((N, d // 2), jnp.float32),
        ],
    )(pre, acc, g, be)


def _run_k2(s1lo, s1hi, hlo, hhi, dinv, b1, g1, be1):
    row = pl.BlockSpec((BLK, DH2), lambda j: (j, 0))
    vec = pl.BlockSpec((1, DHID), lambda j: (0, 0))
    pre, acc = pl.pallas_call(
        _k2a_body,
        grid=(NB,),
        in_specs=[row, row, row, row,
                  pl.BlockSpec((BLK, 1), lambda j: (j, 0)), vec],
        out_specs=[pl.BlockSpec((BLK, DHID), lambda j: (j, 0)),
                   pl.BlockSpec((2, DHID), lambda j: (0, 0))],
        out_shape=[
            jax.ShapeDtypeStruct((N, DHID), jnp.float32),
            jax.ShapeDtypeStruct((2, DHID), jnp.float32),
        ],
    )(s1lo, s1hi, hlo, hhi, dinv, b1)
    return _run_bn(pre, acc, g1, be1)


def _k3a_body(s2lo_ref, s2hi_ref, h2lo_ref, h2hi_ref, cnti_ref, wl_ref,
              wr_ref, bl_ref, pre_ref, acc_ref):
    j = pl.program_id(0)
    s2 = jnp.concatenate([s2lo_ref[...], s2hi_ref[...]], axis=1)
    h2 = jnp.concatenate([h2lo_ref[...], h2hi_ref[...]], axis=1)
    agg = s2 * cnti_ref[...]
    pre = (jnp.dot(agg, wl_ref[...], preferred_element_type=jnp.float32)
           + jnp.dot(h2, wr_ref[...], preferred_element_type=jnp.float32)
           + bl_ref[...])
    pre_ref[...] = pre

    @pl.when(j == 0)
    def _():
        acc_ref[...] = jnp.zeros_like(acc_ref)
    acc_ref[0, :] += jnp.sum(pre, axis=0)
    acc_ref[1, :] += jnp.sum(pre * pre, axis=0)


def _run_k3(s2lo, s2hi, h2lo, h2hi, cnti, Wl, Wr, bl, g2, be2):
    row = pl.BlockSpec((BLK, DH2), lambda j: (j, 0))
    vec = pl.BlockSpec((1, DOUT), lambda j: (0, 0))
    wspec = pl.BlockSpec((DHID, DOUT), lambda j: (0, 0))
    pre, acc = pl.pallas_call(
        _k3a_body,
        grid=(NB,),
        in_specs=[row, row, row, row,
                  pl.BlockSpec((BLK, 1), lambda j: (j, 0)),
                  wspec, wspec, vec],
        out_specs=[pl.BlockSpec((BLK, DOUT), lambda j: (j, 0)),
                   pl.BlockSpec((2, DOUT), lambda j: (0, 0))],
        out_shape=[
            jax.ShapeDtypeStruct((N, DOUT), jnp.float32),
            jax.ShapeDtypeStruct((2, DOUT), jnp.float32),
        ],
    )(s2lo, s2hi, h2lo, h2hi, cnti, Wl, Wr, bl)
    lo, hi = _run_bn(pre, acc, g2, be2)
    return jnp.concatenate([lo, hi], axis=1)


def _deg_parts(dst):
    deg = jnp.zeros((N,), jnp.float32).at[dst].add(1.0)
    return jnp.stack([deg, jnp.zeros((N,), jnp.float32)], axis=1)


def _scatter_rows(src, dst, tlo, thi):
    slo = jnp.zeros((N, DH2), jnp.float32).at[dst].add(tlo[src])
    shi = jnp.zeros((N, DH2), jnp.float32).at[dst].add(thi[src])
    return slo, shi


def kernel(x, edge_index, W1, b1, g1, beta1, Wl, bl, Wr, g2, beta2):
    src = edge_index[0].astype(jnp.int32)
    dst = edge_index[1].astype(jnp.int32)

    degp = _deg_parts(dst)
    hlo, hhi, dinv, cnti = _run_k1(x, W1, degp)
    s1lo, s1hi = _scatter_rows(src, dst, hlo, hhi)
    h2lo, h2hi = _run_k2(s1lo, s1hi, hlo, hhi, dinv,
                         b1.reshape(1, DHID), g1.reshape(1, DHID),
                         beta1.reshape(1, DHID))
    s2lo, s2hi = _scatter_rows(src, dst, h2lo, h2hi)
    out = _run_k3(s2lo, s2hi, h2lo, h2hi, cnti, Wl, Wr,
                  bl.reshape(1, DOUT), g2.reshape(1, DOUT),
                  beta2.reshape(1, DOUT))
    return out


# speedup vs baseline: 2.3803x; 1.2231x over previous
import functools

import jax
import jax.numpy as jnp
from jax import lax
from jax.experimental import pallas as pl
from jax.experimental.pallas import tpu as pltpu
from jax.experimental.pallas import tpu_sc as plsc

N = 10000
E = 160000
DIN = 256
DHID = 512
DOUT = 256

BLK = 400
NB = N // BLK

_NC = 2
_NS = 16
_NW = _NC * _NS
_EPW = E // _NW


def _k1_body(x_ref, w_ref, degp_ref, hs_ref, dinv_ref, cnti_ref):
    h = jnp.dot(x_ref[...], w_ref[...], preferred_element_type=jnp.float32)
    deg = jnp.sum(degp_ref[...], axis=1) + 1.0
    dinv = jax.lax.rsqrt(deg)
    hs_ref[...] = h * dinv[:, None]
    dinv_ref[...] = dinv[:, None]
    cnti_ref[...] = (1.0 / jnp.maximum(deg - 1.0, 1.0))[:, None]


def _run_k1(x, W1, degp):
    return pl.pallas_call(
        _k1_body,
        grid=(NB,),
        in_specs=[
            pl.BlockSpec((BLK, DIN), lambda i: (i, 0)),
            pl.BlockSpec((DIN, DHID), lambda i: (0, 0)),
            pl.BlockSpec((BLK, _NW), lambda i: (i, 0)),
        ],
        out_specs=[
            pl.BlockSpec((BLK, DHID), lambda i: (i, 0)),
            pl.BlockSpec((BLK, 1), lambda i: (i, 0)),
            pl.BlockSpec((BLK, 1), lambda i: (i, 0)),
        ],
        out_shape=[
            jax.ShapeDtypeStruct((N, DHID), jnp.float32),
            jax.ShapeDtypeStruct((N, 1), jnp.float32),
            jax.ShapeDtypeStruct((N, 1), jnp.float32),
        ],
    )(x, W1, degp)


def _k2a_body(s1_ref, hs_ref, dinv_ref, b1_ref, pre_ref, acc_ref):
    j = pl.program_id(0)
    pre = dinv_ref[...] * (s1_ref[...] + hs_ref[...]) + b1_ref[...]
    pre_ref[...] = pre

    @pl.when(j == 0)
    def _():
        acc_ref[...] = jnp.zeros_like(acc_ref)
    acc_ref[0, :] += jnp.sum(pre, axis=0)
    acc_ref[1, :] += jnp.sum(pre * pre, axis=0)


def _bn_body(pre_ref, acc_ref, g_ref, be_ref, out_ref):
    mean = acc_ref[0, :] / N
    var = acc_ref[1, :] / N - mean * mean
    h = (pre_ref[...] - mean) * jax.lax.rsqrt(var + 1e-5) * g_ref[0, :] \
        + be_ref[0, :]
    out_ref[...] = jnp.maximum(h, 0.0)


def _run_bn(pre, acc, g, be):
    d = pre.shape[1]
    vec = pl.BlockSpec((1, d), lambda j: (0, 0))
    return pl.pallas_call(
        _bn_body,
        grid=(NB,),
        in_specs=[pl.BlockSpec((BLK, d), lambda j: (j, 0)),
                  pl.BlockSpec((2, d), lambda j: (0, 0)),
                  vec, vec],
        out_specs=pl.BlockSpec((BLK, d), lambda j: (j, 0)),
        out_shape=jax.ShapeDtypeStruct((N, d), jnp.float32),
    )(pre, acc, g, be)


def _run_k2(s1, hs, dinv, b1, g1, be1):
    row = pl.BlockSpec((BLK, DHID), lambda j: (j, 0))
    vec = pl.BlockSpec((1, DHID), lambda j: (0, 0))
    pre, acc = pl.pallas_call(
        _k2a_body,
        grid=(NB,),
        in_specs=[row, row,
                  pl.BlockSpec((BLK, 1), lambda j: (j, 0)), vec],
        out_specs=[row, pl.BlockSpec((2, DHID), lambda j: (0, 0))],
        out_shape=[
            jax.ShapeDtypeStruct((N, DHID), jnp.float32),
            jax.ShapeDtypeStruct((2, DHID), jnp.float32),
        ],
    )(s1, hs, dinv, b1)
    return _run_bn(pre, acc, g1, be1)


def _k3a_body(s2_ref, h2_ref, cnti_ref, wl_ref, wr_ref, bl_ref,
              pre_ref, acc_ref):
    j = pl.program_id(0)
    agg = s2_ref[...] * cnti_ref[...]
    pre = (jnp.dot(agg, wl_ref[...], preferred_element_type=jnp.float32)
           + jnp.dot(h2_ref[...], wr_ref[...],
                     preferred_element_type=jnp.float32)
           + bl_ref[...])
    pre_ref[...] = pre

    @pl.when(j == 0)
    def _():
        acc_ref[...] = jnp.zeros_like(acc_ref)
    acc_ref[0, :] += jnp.sum(pre, axis=0)
    acc_ref[1, :] += jnp.sum(pre * pre, axis=0)


def _run_k3(s2, h2, cnti, Wl, Wr, bl, g2, be2):
    row = pl.BlockSpec((BLK, DHID), lambda j: (j, 0))
    vec = pl.BlockSpec((1, DOUT), lambda j: (0, 0))
    wspec = pl.BlockSpec((DHID, DOUT), lambda j: (0, 0))
    pre, acc = pl.pallas_call(
        _k3a_body,
        grid=(NB,),
        in_specs=[row, row,
                  pl.BlockSpec((BLK, 1), lambda j: (j, 0)),
                  wspec, wspec, vec],
        out_specs=[pl.BlockSpec((BLK, DOUT), lambda j: (j, 0)),
                   pl.BlockSpec((2, DOUT), lambda j: (0, 0))],
        out_shape=[
            jax.ShapeDtypeStruct((N, DOUT), jnp.float32),
            jax.ShapeDtypeStruct((2, DOUT), jnp.float32),
        ],
    )(s2, h2, cnti, Wl, Wr, bl)
    return _run_bn(pre, acc, g2, be2)


def _deg_sc_body(dst_hbm, out_hbm, dst_v, hist_v):
    c = lax.axis_index("c")
    s = lax.axis_index("s")
    wid = s * _NC + c

    pltpu.sync_copy(dst_hbm.at[pl.ds(wid * _EPW, _EPW)],
                    dst_v.at[pl.ds(0, _EPW)])

    def zbody(i, carry):
        hist_v[pl.ds(i * 16, 16)] = jnp.zeros((16,), jnp.float32)
        return carry
    lax.fori_loop(0, N // 16, zbody, 0)

    ones = jnp.full((16,), 1.0, jnp.float32)
    nfull = _EPW // 16

    def body(k, carry):
        idx = dst_v[pl.ds(k * 16, 16)]
        plsc.addupdate_scatter(hist_v, [idx], ones)
        return carry
    lax.fori_loop(0, nfull, body, 0)

    rem = _EPW - nfull * 16
    if rem:
        lane = lax.broadcasted_iota(jnp.int32, (16,), 0)
        idx = dst_v[pl.ds(nfull * 16, 16)]
        plsc.addupdate_scatter(hist_v, [idx], ones, mask=lane < rem)

    pltpu.sync_copy(hist_v, out_hbm.at[wid])


def _sc_degree(dst):
    mesh = plsc.VectorSubcoreMesh(core_axis_name="c", subcore_axis_name="s")
    return pl.kernel(
        _deg_sc_body,
        out_type=jax.ShapeDtypeStruct((_NW, N), jnp.float32),
        mesh=mesh,
        scratch_types=[
            pltpu.VMEM((_EPW + 16,), jnp.int32),
            pltpu.VMEM((N,), jnp.float32),
        ],
        compiler_params=pltpu.CompilerParams(needs_layout_passes=False),
    )(dst)


_CPT = 4
_NG = DHID // _CPT
_NP = _NG // _NW
_EW = 4000
_NWIN = E // _EW


def _scat_sc_body(src_hbm, dst_hbm, tab_hbm, z_hbm, out_hbm,
                  stage, acc, sbuf, dbuf):
    c = lax.axis_index("c")
    s = lax.axis_index("s")
    wid = s * _NC + c

    for p in range(_NP):
        g = p * _NW + wid
        pltpu.sync_copy(tab_hbm.at[g], stage)
        pltpu.sync_copy(z_hbm, acc)

        def win(w, carry):
            pltpu.sync_copy(src_hbm.at[pl.ds(w * _EW, _EW)], sbuf)
            pltpu.sync_copy(dst_hbm.at[pl.ds(w * _EW, _EW)], dbuf)

            def chunk(k, carry2):
                si = sbuf[pl.ds(k * 16, 16)]
                di = dbuf[pl.ds(k * 16, 16)]
                for cc in range(_CPT):
                    rv = jnp.full((16,), cc, jnp.int32)
                    v = plsc.load_gather(stage, [rv, si])
                    plsc.addupdate_scatter(acc, [rv, di], v)
                return carry2
            lax.fori_loop(0, _EW // 16, chunk, 0)
            return carry
        lax.fori_loop(0, _NWIN, win, 0)

        pltpu.sync_copy(acc, out_hbm.at[g])


def _sc_scatter(src, dst, tab_t3, z4):
    mesh = plsc.VectorSubcoreMesh(core_axis_name="c", subcore_axis_name="s")
    return pl.kernel(
        _scat_sc_body,
        out_type=jax.ShapeDtypeStruct((_NG, _CPT, N), jnp.float32),
        mesh=mesh,
        scratch_types=[
            pltpu.VMEM((_CPT, N), jnp.float32),
            pltpu.VMEM((_CPT, N), jnp.float32),
            pltpu.VMEM((_EW,), jnp.int32),
            pltpu.VMEM((_EW,), jnp.int32),
        ],
        compiler_params=pltpu.CompilerParams(needs_layout_passes=False),
    )(src, dst, tab_t3, z4)


def kernel(x, edge_index, W1, b1, g1, beta1, Wl, bl, Wr, g2, beta2):
    src = edge_index[0].astype(jnp.int32)
    dst = edge_index[1].astype(jnp.int32)
    z4 = jnp.zeros((_CPT, N), jnp.float32)

    degp = _sc_degree(dst).T
    hs, dinv, cnti = _run_k1(x, W1, degp)
    hs_t3 = hs.T.reshape(_NG, _CPT, N)
    s1 = _sc_scatter(src, dst, hs_t3, z4).reshape(DHID, N).T
    h2 = _run_k2(s1, hs, dinv, b1.reshape(1, DHID),
                 g1.reshape(1, DHID), beta1.reshape(1, DHID))
    h2_t3 = h2.T.reshape(_NG, _CPT, N)
    s2 = _sc_scatter(src, dst, h2_t3, z4).reshape(DHID, N).T
    out = _run_k3(s2, h2, cnti, Wl, Wr, bl.reshape(1, DOUT),
                  g2.reshape(1, DOUT), beta2.reshape(1, DOUT))
    return out


# speedup vs baseline: 2.7948x; 1.1742x over previous
import functools

import jax
import jax.numpy as jnp
from jax import lax
from jax.experimental import pallas as pl
from jax.experimental.pallas import tpu as pltpu
from jax.experimental.pallas import tpu_sc as plsc

N = 10000
E = 160000
DIN = 256
DHID = 512
DOUT = 256

BLK = 400
NB = N // BLK

_NC = 2
_NS = 16
_NW = _NC * _NS
_EPW = E // _NW


def _k1_body(x_ref, w_ref, degp_ref, hs_ref, dinv_ref, cnti_ref):
    h = jnp.dot(x_ref[...], w_ref[...], preferred_element_type=jnp.float32)
    deg = jnp.sum(degp_ref[...], axis=1) + 1.0
    dinv = jax.lax.rsqrt(deg)
    hs_ref[...] = h * dinv[:, None]
    dinv_ref[...] = dinv[:, None]
    cnti_ref[...] = (1.0 / jnp.maximum(deg - 1.0, 1.0))[:, None]


def _run_k1(x, W1, degp):
    return pl.pallas_call(
        _k1_body,
        grid=(NB,),
        in_specs=[
            pl.BlockSpec((BLK, DIN), lambda i: (i, 0)),
            pl.BlockSpec((DIN, DHID), lambda i: (0, 0)),
            pl.BlockSpec((BLK, _NW), lambda i: (i, 0)),
        ],
        out_specs=[
            pl.BlockSpec((BLK, DHID), lambda i: (i, 0)),
            pl.BlockSpec((BLK, 1), lambda i: (i, 0)),
            pl.BlockSpec((BLK, 1), lambda i: (i, 0)),
        ],
        out_shape=[
            jax.ShapeDtypeStruct((N, DHID), jnp.float32),
            jax.ShapeDtypeStruct((N, 1), jnp.float32),
            jax.ShapeDtypeStruct((N, 1), jnp.float32),
        ],
    )(x, W1, degp)


def _k2a_body(s1_ref, hs_ref, dinv_ref, b1_ref, pre_ref, acc_ref):
    j = pl.program_id(0)
    pre = dinv_ref[...] * (s1_ref[...] + hs_ref[...]) + b1_ref[...]
    pre_ref[...] = pre

    @pl.when(j == 0)
    def _():
        acc_ref[...] = jnp.zeros_like(acc_ref)
    acc_ref[0, :] += jnp.sum(pre, axis=0)
    acc_ref[1, :] += jnp.sum(pre * pre, axis=0)


def _bn_body(pre_ref, acc_ref, g_ref, be_ref, out_ref):
    mean = acc_ref[0, :] / N
    var = acc_ref[1, :] / N - mean * mean
    h = (pre_ref[...] - mean) * jax.lax.rsqrt(var + 1e-5) * g_ref[0, :] \
        + be_ref[0, :]
    out_ref[...] = jnp.maximum(h, 0.0)


def _run_bn(pre, acc, g, be):
    d = pre.shape[1]
    vec = pl.BlockSpec((1, d), lambda j: (0, 0))
    return pl.pallas_call(
        _bn_body,
        grid=(NB,),
        in_specs=[pl.BlockSpec((BLK, d), lambda j: (j, 0)),
                  pl.BlockSpec((2, d), lambda j: (0, 0)),
                  vec, vec],
        out_specs=pl.BlockSpec((BLK, d), lambda j: (j, 0)),
        out_shape=jax.ShapeDtypeStruct((N, d), jnp.float32),
    )(pre, acc, g, be)


def _run_k2(s1, hs, dinv, b1, g1, be1):
    row = pl.BlockSpec((BLK, DHID), lambda j: (j, 0))
    vec = pl.BlockSpec((1, DHID), lambda j: (0, 0))
    pre, acc = pl.pallas_call(
        _k2a_body,
        grid=(NB,),
        in_specs=[row, row,
                  pl.BlockSpec((BLK, 1), lambda j: (j, 0)), vec],
        out_specs=[row, pl.BlockSpec((2, DHID), lambda j: (0, 0))],
        out_shape=[
            jax.ShapeDtypeStruct((N, DHID), jnp.float32),
            jax.ShapeDtypeStruct((2, DHID), jnp.float32),
        ],
    )(s1, hs, dinv, b1)
    return _run_bn(pre, acc, g1, be1)


def _k3a_body(s2_ref, h2_ref, cnti_ref, wl_ref, wr_ref, bl_ref,
              pre_ref, acc_ref):
    j = pl.program_id(0)
    agg = s2_ref[...] * cnti_ref[...]
    pre = (jnp.dot(agg, wl_ref[...], preferred_element_type=jnp.float32)
           + jnp.dot(h2_ref[...], wr_ref[...],
                     preferred_element_type=jnp.float32)
           + bl_ref[...])
    pre_ref[...] = pre

    @pl.when(j == 0)
    def _():
        acc_ref[...] = jnp.zeros_like(acc_ref)
    acc_ref[0, :] += jnp.sum(pre, axis=0)
    acc_ref[1, :] += jnp.sum(pre * pre, axis=0)


def _run_k3(s2, h2, cnti, Wl, Wr, bl, g2, be2):
    row = pl.BlockSpec((BLK, DHID), lambda j: (j, 0))
    vec = pl.BlockSpec((1, DOUT), lambda j: (0, 0))
    wspec = pl.BlockSpec((DHID, DOUT), lambda j: (0, 0))
    pre, acc = pl.pallas_call(
        _k3a_body,
        grid=(NB,),
        in_specs=[row, row,
                  pl.BlockSpec((BLK, 1), lambda j: (j, 0)),
                  wspec, wspec, vec],
        out_specs=[pl.BlockSpec((BLK, DOUT), lambda j: (j, 0)),
                   pl.BlockSpec((2, DOUT), lambda j: (0, 0))],
        out_shape=[
            jax.ShapeDtypeStruct((N, DOUT), jnp.float32),
            jax.ShapeDtypeStruct((2, DOUT), jnp.float32),
        ],
    )(s2, h2, cnti, Wl, Wr, bl)
    return _run_bn(pre, acc, g2, be2)


def _deg_sc_body(dst_hbm, out_hbm, dst_v, hist_v):
    c = lax.axis_index("c")
    s = lax.axis_index("s")
    wid = s * _NC + c

    pltpu.sync_copy(dst_hbm.at[pl.ds(wid * _EPW, _EPW)],
                    dst_v.at[pl.ds(0, _EPW)])

    def zbody(i, carry):
        hist_v[pl.ds(i * 16, 16)] = jnp.zeros((16,), jnp.float32)
        return carry
    lax.fori_loop(0, N // 16, zbody, 0)

    ones = jnp.full((16,), 1.0, jnp.float32)
    nfull = _EPW // 16

    def body(k, carry):
        idx = dst_v[pl.ds(k * 16, 16)]
        plsc.addupdate_scatter(hist_v, [idx], ones)
        return carry
    lax.fori_loop(0, nfull, body, 0)

    rem = _EPW - nfull * 16
    if rem:
        lane = lax.broadcasted_iota(jnp.int32, (16,), 0)
        idx = dst_v[pl.ds(nfull * 16, 16)]
        plsc.addupdate_scatter(hist_v, [idx], ones, mask=lane < rem)

    pltpu.sync_copy(hist_v, out_hbm.at[wid])


def _sc_degree(dst):
    mesh = plsc.VectorSubcoreMesh(core_axis_name="c", subcore_axis_name="s")
    return pl.kernel(
        _deg_sc_body,
        out_type=jax.ShapeDtypeStruct((_NW, N), jnp.float32),
        mesh=mesh,
        scratch_types=[
            pltpu.VMEM((_EPW + 16,), jnp.int32),
            pltpu.VMEM((N,), jnp.float32),
        ],
        compiler_params=pltpu.CompilerParams(needs_layout_passes=False),
    )(dst)


_CPT = 4
_NG = DHID // _CPT
_NP = _NG // _NW
_EW = 4000
_NWIN = E // _EW
_UNR = 5


def _scat_sc_body(src_hbm, dst_hbm, tab_hbm, z_hbm, out_hbm,
                  stage, acc, sbuf, dbuf):
    c = lax.axis_index("c")
    s = lax.axis_index("s")
    wid = s * _NC + c

    for p in range(_NP):
        g = p * _NW + wid
        pltpu.sync_copy(tab_hbm.at[g], stage)
        pltpu.sync_copy(z_hbm, acc)

        def win(w, carry):
            pltpu.sync_copy(src_hbm.at[pl.ds(w * _EW, _EW)], sbuf)
            pltpu.sync_copy(dst_hbm.at[pl.ds(w * _EW, _EW)], dbuf)

            def chunk(k, carry2):
                sis = [sbuf[pl.ds((k * _UNR + u) * 16, 16)]
                       for u in range(_UNR)]
                dis = [dbuf[pl.ds((k * _UNR + u) * 16, 16)]
                       for u in range(_UNR)]
                for u in range(_UNR):
                    for cc in range(_CPT):
                        rv = jnp.full((16,), cc, jnp.int32)
                        v = plsc.load_gather(stage, [rv, sis[u]])
                        plsc.addupdate_scatter(acc, [rv, dis[u]], v)
                return carry2
            lax.fori_loop(0, _EW // 16 // _UNR, chunk, 0)
            return carry
        lax.fori_loop(0, _NWIN, win, 0)

        pltpu.sync_copy(acc, out_hbm.at[g])


def _sc_scatter(src, dst, tab_t3, z4):
    mesh = plsc.VectorSubcoreMesh(core_axis_name="c", subcore_axis_name="s")
    return pl.kernel(
        _scat_sc_body,
        out_type=jax.ShapeDtypeStruct((_NG, _CPT, N), jnp.float32),
        mesh=mesh,
        scratch_types=[
            pltpu.VMEM((_CPT, N), jnp.float32),
            pltpu.VMEM((_CPT, N), jnp.float32),
            pltpu.VMEM((_EW,), jnp.int32),
            pltpu.VMEM((_EW,), jnp.int32),
        ],
        compiler_params=pltpu.CompilerParams(needs_layout_passes=False),
    )(src, dst, tab_t3, z4)


def kernel(x, edge_index, W1, b1, g1, beta1, Wl, bl, Wr, g2, beta2):
    src = edge_index[0].astype(jnp.int32)
    dst = edge_index[1].astype(jnp.int32)
    z4 = jnp.zeros((_CPT, N), jnp.float32)

    degp = _sc_degree(dst).T
    hs, dinv, cnti = _run_k1(x, W1, degp)
    hs_t3 = hs.T.reshape(_NG, _CPT, N)
    s1 = _sc_scatter(src, dst, hs_t3, z4).reshape(DHID, N).T
    h2 = _run_k2(s1, hs, dinv, b1.reshape(1, DHID),
                 g1.reshape(1, DHID), beta1.reshape(1, DHID))
    h2_t3 = h2.T.reshape(_NG, _CPT, N)
    s2 = _sc_scatter(src, dst, h2_t3, z4).reshape(DHID, N).T
    out = _run_k3(s2, h2, cnti, Wl, Wr, bl.reshape(1, DOUT),
                  g2.reshape(1, DOUT), beta2.reshape(1, DOUT))
    return out


# speedup vs baseline: 2.8593x; 1.0231x over previous
import functools

import jax
import jax.numpy as jnp
from jax import lax
from jax.experimental import pallas as pl
from jax.experimental.pallas import tpu as pltpu
from jax.experimental.pallas import tpu_sc as plsc

N = 10000
E = 160000
DIN = 256
DHID = 512
DOUT = 256

BLK = 400
NB = N // BLK

_NC = 2
_NS = 16
_NW = _NC * _NS
_EPW = E // _NW


def _k1_body(x_ref, w_ref, degp_ref, hs_ref, dinv_ref, cnti_ref):
    h = jnp.dot(x_ref[...], w_ref[...], preferred_element_type=jnp.float32)
    deg = jnp.sum(degp_ref[...], axis=1) + 1.0
    dinv = jax.lax.rsqrt(deg)
    hs_ref[...] = h * dinv[:, None]
    dinv_ref[...] = dinv[:, None]
    cnti_ref[...] = (1.0 / jnp.maximum(deg - 1.0, 1.0))[:, None]


def _run_k1(x, W1, degp):
    return pl.pallas_call(
        _k1_body,
        grid=(NB,),
        in_specs=[
            pl.BlockSpec((BLK, DIN), lambda i: (i, 0)),
            pl.BlockSpec((DIN, DHID), lambda i: (0, 0)),
            pl.BlockSpec((BLK, _NW), lambda i: (i, 0)),
        ],
        out_specs=[
            pl.BlockSpec((BLK, DHID), lambda i: (i, 0)),
            pl.BlockSpec((BLK, 1), lambda i: (i, 0)),
            pl.BlockSpec((BLK, 1), lambda i: (i, 0)),
        ],
        out_shape=[
            jax.ShapeDtypeStruct((N, DHID), jnp.float32),
            jax.ShapeDtypeStruct((N, 1), jnp.float32),
            jax.ShapeDtypeStruct((N, 1), jnp.float32),
        ],
    )(x, W1, degp)


def _k2a_body(s1_ref, hs_ref, dinv_ref, b1_ref, pre_ref, acc_ref):
    j = pl.program_id(0)
    pre = dinv_ref[...] * (s1_ref[...] + hs_ref[...]) + b1_ref[...]
    pre_ref[...] = pre

    @pl.when(j == 0)
    def _():
        acc_ref[...] = jnp.zeros_like(acc_ref)
    acc_ref[0, :] += jnp.sum(pre, axis=0)
    acc_ref[1, :] += jnp.sum(pre * pre, axis=0)


def _bn_body(pre_ref, acc_ref, g_ref, be_ref, out_ref):
    mean = acc_ref[0, :] / N
    var = acc_ref[1, :] / N - mean * mean
    h = (pre_ref[...] - mean) * jax.lax.rsqrt(var + 1e-5) * g_ref[0, :] \
        + be_ref[0, :]
    out_ref[...] = jnp.maximum(h, 0.0)


def _run_bn(pre, acc, g, be):
    d = pre.shape[1]
    vec = pl.BlockSpec((1, d), lambda j: (0, 0))
    return pl.pallas_call(
        _bn_body,
        grid=(NB,),
        in_specs=[pl.BlockSpec((BLK, d), lambda j: (j, 0)),
                  pl.BlockSpec((2, d), lambda j: (0, 0)),
                  vec, vec],
        out_specs=pl.BlockSpec((BLK, d), lambda j: (j, 0)),
        out_shape=jax.ShapeDtypeStruct((N, d), jnp.float32),
    )(pre, acc, g, be)


def _run_k2(s1, hs, dinv, b1, g1, be1):
    row = pl.BlockSpec((BLK, DHID), lambda j: (j, 0))
    vec = pl.BlockSpec((1, DHID), lambda j: (0, 0))
    pre, acc = pl.pallas_call(
        _k2a_body,
        grid=(NB,),
        in_specs=[row, row,
                  pl.BlockSpec((BLK, 1), lambda j: (j, 0)), vec],
        out_specs=[row, pl.BlockSpec((2, DHID), lambda j: (0, 0))],
        out_shape=[
            jax.ShapeDtypeStruct((N, DHID), jnp.float32),
            jax.ShapeDtypeStruct((2, DHID), jnp.float32),
        ],
    )(s1, hs, dinv, b1)
    return _run_bn(pre, acc, g1, be1)


def _k3a_body(s2_ref, h2_ref, cnti_ref, wl_ref, wr_ref, bl_ref,
              pre_ref, acc_ref):
    j = pl.program_id(0)
    agg = s2_ref[...] * cnti_ref[...]
    pre = (jnp.dot(agg, wl_ref[...], preferred_element_type=jnp.float32)
           + jnp.dot(h2_ref[...], wr_ref[...],
                     preferred_element_type=jnp.float32)
           + bl_ref[...])
    pre_ref[...] = pre

    @pl.when(j == 0)
    def _():
        acc_ref[...] = jnp.zeros_like(acc_ref)
    acc_ref[0, :] += jnp.sum(pre, axis=0)
    acc_ref[1, :] += jnp.sum(pre * pre, axis=0)


def _run_k3(s2, h2, cnti, Wl, Wr, bl, g2, be2):
    row = pl.BlockSpec((BLK, DHID), lambda j: (j, 0))
    vec = pl.BlockSpec((1, DOUT), lambda j: (0, 0))
    wspec = pl.BlockSpec((DHID, DOUT), lambda j: (0, 0))
    pre, acc = pl.pallas_call(
        _k3a_body,
        grid=(NB,),
        in_specs=[row, row,
                  pl.BlockSpec((BLK, 1), lambda j: (j, 0)),
                  wspec, wspec, vec],
        out_specs=[pl.BlockSpec((BLK, DOUT), lambda j: (j, 0)),
                   pl.BlockSpec((2, DOUT), lambda j: (0, 0))],
        out_shape=[
            jax.ShapeDtypeStruct((N, DOUT), jnp.float32),
            jax.ShapeDtypeStruct((2, DOUT), jnp.float32),
        ],
    )(s2, h2, cnti, Wl, Wr, bl)
    return _run_bn(pre, acc, g2, be2)


def _deg_sc_body(dst_hbm, out_hbm, dst_v, hist_v):
    c = lax.axis_index("c")
    s = lax.axis_index("s")
    wid = s * _NC + c

    pltpu.sync_copy(dst_hbm.at[pl.ds(wid * _EPW, _EPW)],
                    dst_v.at[pl.ds(0, _EPW)])

    def zbody(i, carry):
        hist_v[pl.ds(i * 16, 16)] = jnp.zeros((16,), jnp.float32)
        return carry
    lax.fori_loop(0, N // 16, zbody, 0)

    ones = jnp.full((16,), 1.0, jnp.float32)
    nfull = _EPW // 16

    def body(k, carry):
        idx = dst_v[pl.ds(k * 16, 16)]
        plsc.addupdate_scatter(hist_v, [idx], ones)
        return carry
    lax.fori_loop(0, nfull, body, 0)

    rem = _EPW - nfull * 16
    if rem:
        lane = lax.broadcasted_iota(jnp.int32, (16,), 0)
        idx = dst_v[pl.ds(nfull * 16, 16)]
        plsc.addupdate_scatter(hist_v, [idx], ones, mask=lane < rem)

    pltpu.sync_copy(hist_v, out_hbm.at[wid])


def _sc_degree(dst):
    mesh = plsc.VectorSubcoreMesh(core_axis_name="c", subcore_axis_name="s")
    return pl.kernel(
        _deg_sc_body,
        out_type=jax.ShapeDtypeStruct((_NW, N), jnp.float32),
        mesh=mesh,
        scratch_types=[
            pltpu.VMEM((_EPW + 16,), jnp.int32),
            pltpu.VMEM((N,), jnp.float32),
        ],
        compiler_params=pltpu.CompilerParams(needs_layout_passes=False),
    )(dst)


_CPT = 4
_NG = DHID // _CPT
_NP = _NG // _NW
_EW = 4000
_NWIN = E // _EW
_UNR = 10


def _scat_sc_body(src_hbm, dst_hbm, tab_hbm, z_hbm, out_hbm,
                  stage, acc, sbuf, dbuf):
    c = lax.axis_index("c")
    s = lax.axis_index("s")
    wid = s * _NC + c

    for p in range(_NP):
        g = p * _NW + wid
        pltpu.sync_copy(tab_hbm.at[g], stage)
        pltpu.sync_copy(z_hbm, acc)

        def win(w, carry):
            pltpu.sync_copy(src_hbm.at[pl.ds(w * _EW, _EW)], sbuf)
            pltpu.sync_copy(dst_hbm.at[pl.ds(w * _EW, _EW)], dbuf)

            def chunk(k, carry2):
                sis = [sbuf[pl.ds((k * _UNR + u) * 16, 16)]
                       for u in range(_UNR)]
                dis = [dbuf[pl.ds((k * _UNR + u) * 16, 16)]
                       for u in range(_UNR)]
                for u in range(_UNR):
                    for cc in range(_CPT):
                        rv = jnp.full((16,), cc, jnp.int32)
                        v = plsc.load_gather(stage, [rv, sis[u]])
                        plsc.addupdate_scatter(acc, [rv, dis[u]], v)
                return carry2
            lax.fori_loop(0, _EW // 16 // _UNR, chunk, 0)
            return carry
        lax.fori_loop(0, _NWIN, win, 0)

        pltpu.sync_copy(acc, out_hbm.at[g])


def _sc_scatter(src, dst, tab_t3, z4):
    mesh = plsc.VectorSubcoreMesh(core_axis_name="c", subcore_axis_name="s")
    return pl.kernel(
        _scat_sc_body,
        out_type=jax.ShapeDtypeStruct((_NG, _CPT, N), jnp.float32),
        mesh=mesh,
        scratch_types=[
            pltpu.VMEM((_CPT, N), jnp.float32),
            pltpu.VMEM((_CPT, N), jnp.float32),
            pltpu.VMEM((_EW,), jnp.int32),
            pltpu.VMEM((_EW,), jnp.int32),
        ],
        compiler_params=pltpu.CompilerParams(needs_layout_passes=False),
    )(src, dst, tab_t3, z4)


def kernel(x, edge_index, W1, b1, g1, beta1, Wl, bl, Wr, g2, beta2):
    src = edge_index[0].astype(jnp.int32)
    dst = edge_index[1].astype(jnp.int32)
    z4 = jnp.zeros((_CPT, N), jnp.float32)

    degp = _sc_degree(dst).T
    hs, dinv, cnti = _run_k1(x, W1, degp)
    hs_t3 = hs.T.reshape(_NG, _CPT, N)
    s1 = _sc_scatter(src, dst, hs_t3, z4).reshape(DHID, N).T
    h2 = _run_k2(s1, hs, dinv, b1.reshape(1, DHID),
                 g1.reshape(1, DHID), beta1.reshape(1, DHID))
    h2_t3 = h2.T.reshape(_NG, _CPT, N)
    s2 = _sc_scatter(src, dst, h2_t3, z4).reshape(DHID, N).T
    out = _run_k3(s2, h2, cnti, Wl, Wr, bl.reshape(1, DOUT),
                  g2.reshape(1, DOUT), beta2.reshape(1, DOUT))
    return out


# speedup vs baseline: 3.4925x; 1.2214x over previous
import functools

import jax
import jax.numpy as jnp
from jax import lax
from jax.experimental import pallas as pl
from jax.experimental.pallas import tpu as pltpu
from jax.experimental.pallas import tpu_sc as plsc

N = 10000
E = 160000
DIN = 256
DHID = 512
DOUT = 256

BLK = 400
NB = N // BLK

_NC = 2
_NS = 16
_NW = _NC * _NS
_EPW = E // _NW


def _k1_body(x_ref, w_ref, degp_ref, hs_ref, dinv_ref, cnti_ref):
    h = jnp.dot(x_ref[...], w_ref[...], preferred_element_type=jnp.float32)
    deg = jnp.sum(degp_ref[...], axis=1) + 1.0
    dinv = jax.lax.rsqrt(deg)
    hs_ref[...] = h * dinv[:, None]
    dinv_ref[...] = dinv[:, None]
    cnti_ref[...] = (1.0 / jnp.maximum(deg - 1.0, 1.0))[:, None]


def _run_k1(x, W1, degp):
    return pl.pallas_call(
        _k1_body,
        grid=(NB,),
        in_specs=[
            pl.BlockSpec((BLK, DIN), lambda i: (i, 0)),
            pl.BlockSpec((DIN, DHID), lambda i: (0, 0)),
            pl.BlockSpec((BLK, _NW), lambda i: (i, 0)),
        ],
        out_specs=[
            pl.BlockSpec((BLK, DHID), lambda i: (i, 0)),
            pl.BlockSpec((BLK, 1), lambda i: (i, 0)),
            pl.BlockSpec((BLK, 1), lambda i: (i, 0)),
        ],
        out_shape=[
            jax.ShapeDtypeStruct((N, DHID), jnp.float32),
            jax.ShapeDtypeStruct((N, 1), jnp.float32),
            jax.ShapeDtypeStruct((N, 1), jnp.float32),
        ],
    )(x, W1, degp)


def _k2a_body(s1_ref, hs_ref, dinv_ref, b1_ref, pre_ref, acc_ref):
    j = pl.program_id(0)
    pre = dinv_ref[...] * (s1_ref[...] + hs_ref[...]) + b1_ref[...]
    pre_ref[...] = pre

    @pl.when(j == 0)
    def _():
        acc_ref[...] = jnp.zeros_like(acc_ref)
    acc_ref[0, :] += jnp.sum(pre, axis=0)
    acc_ref[1, :] += jnp.sum(pre * pre, axis=0)


def _bn_body(pre_ref, acc_ref, g_ref, be_ref, out_ref):
    mean = acc_ref[0, :] / N
    var = acc_ref[1, :] / N - mean * mean
    h = (pre_ref[...] - mean) * jax.lax.rsqrt(var + 1e-5) * g_ref[0, :] \
        + be_ref[0, :]
    out_ref[...] = jnp.maximum(h, 0.0)


def _run_bn(pre, acc, g, be):
    d = pre.shape[1]
    vec = pl.BlockSpec((1, d), lambda j: (0, 0))
    return pl.pallas_call(
        _bn_body,
        grid=(NB,),
        in_specs=[pl.BlockSpec((BLK, d), lambda j: (j, 0)),
                  pl.BlockSpec((2, d), lambda j: (0, 0)),
                  vec, vec],
        out_specs=pl.BlockSpec((BLK, d), lambda j: (j, 0)),
        out_shape=jax.ShapeDtypeStruct((N, d), jnp.float32),
    )(pre, acc, g, be)


def _run_k2(s1, hs, dinv, b1, g1, be1):
    row = pl.BlockSpec((BLK, DHID), lambda j: (j, 0))
    vec = pl.BlockSpec((1, DHID), lambda j: (0, 0))
    pre, acc = pl.pallas_call(
        _k2a_body,
        grid=(NB,),
        in_specs=[row, row,
                  pl.BlockSpec((BLK, 1), lambda j: (j, 0)), vec],
        out_specs=[row, pl.BlockSpec((2, DHID), lambda j: (0, 0))],
        out_shape=[
            jax.ShapeDtypeStruct((N, DHID), jnp.float32),
            jax.ShapeDtypeStruct((2, DHID), jnp.float32),
        ],
    )(s1, hs, dinv, b1)
    return _run_bn(pre, acc, g1, be1)


def _k3a_body(s2_ref, h2_ref, cnti_ref, wl_ref, wr_ref, bl_ref,
              pre_ref, acc_ref):
    j = pl.program_id(0)
    agg = s2_ref[...] * cnti_ref[...]
    pre = (jnp.dot(agg, wl_ref[...], preferred_element_type=jnp.float32)
           + jnp.dot(h2_ref[...], wr_ref[...],
                     preferred_element_type=jnp.float32)
           + bl_ref[...])
    pre_ref[...] = pre

    @pl.when(j == 0)
    def _():
        acc_ref[...] = jnp.zeros_like(acc_ref)
    acc_ref[0, :] += jnp.sum(pre, axis=0)
    acc_ref[1, :] += jnp.sum(pre * pre, axis=0)


def _run_k3(s2, h2, cnti, Wl, Wr, bl, g2, be2):
    row = pl.BlockSpec((BLK, DHID), lambda j: (j, 0))
    vec = pl.BlockSpec((1, DOUT), lambda j: (0, 0))
    wspec = pl.BlockSpec((DHID, DOUT), lambda j: (0, 0))
    pre, acc = pl.pallas_call(
        _k3a_body,
        grid=(NB,),
        in_specs=[row, row,
                  pl.BlockSpec((BLK, 1), lambda j: (j, 0)),
                  wspec, wspec, vec],
        out_specs=[pl.BlockSpec((BLK, DOUT), lambda j: (j, 0)),
                   pl.BlockSpec((2, DOUT), lambda j: (0, 0))],
        out_shape=[
            jax.ShapeDtypeStruct((N, DOUT), jnp.float32),
            jax.ShapeDtypeStruct((2, DOUT), jnp.float32),
        ],
    )(s2, h2, cnti, Wl, Wr, bl)
    return _run_bn(pre, acc, g2, be2)


def _deg_sc_body(dst_hbm, out_hbm, dst_v, hist_v):
    c = lax.axis_index("c")
    s = lax.axis_index("s")
    wid = s * _NC + c

    pltpu.sync_copy(dst_hbm.at[pl.ds(wid * _EPW, _EPW)],
                    dst_v.at[pl.ds(0, _EPW)])

    def zbody(i, carry):
        hist_v[pl.ds(i * 16, 16)] = jnp.zeros((16,), jnp.float32)
        return carry
    lax.fori_loop(0, N // 16, zbody, 0)

    ones = jnp.full((16,), 1.0, jnp.float32)
    nfull = _EPW // 16

    def body(k, carry):
        idx = dst_v[pl.ds(k * 16, 16)]
        plsc.addupdate_scatter(hist_v, [idx], ones)
        return carry
    lax.fori_loop(0, nfull, body, 0)

    rem = _EPW - nfull * 16
    if rem:
        lane = lax.broadcasted_iota(jnp.int32, (16,), 0)
        idx = dst_v[pl.ds(nfull * 16, 16)]
        plsc.addupdate_scatter(hist_v, [idx], ones, mask=lane < rem)

    pltpu.sync_copy(hist_v, out_hbm.at[wid])


def _sc_degree(dst):
    mesh = plsc.VectorSubcoreMesh(core_axis_name="c", subcore_axis_name="s")
    return pl.kernel(
        _deg_sc_body,
        out_type=jax.ShapeDtypeStruct((_NW, N), jnp.float32),
        mesh=mesh,
        scratch_types=[
            pltpu.VMEM((_EPW + 16,), jnp.int32),
            pltpu.VMEM((N,), jnp.float32),
        ],
        compiler_params=pltpu.CompilerParams(needs_layout_passes=False),
    )(dst)


_CPT = 8
_CPW = _CPT // 2
_NG = DHID // _CPT
_NP = _NG // _NW
_EW = 4000
_NWIN = E // _EW
_UNR = 10


def _scat_sc_body(src_hbm, dst_hbm, tab_hbm, z_hbm, out_hbm,
                  stage, acc, sbuf, dbuf):
    c = lax.axis_index("c")
    s = lax.axis_index("s")
    wid = s * _NC + c
    rvw = [jnp.full((16,), w, jnp.int32) for w in range(_CPW)]
    rvc = [jnp.full((16,), cc, jnp.int32) for cc in range(_CPT)]

    for p in range(_NP):
        g = p * _NW + wid
        pltpu.sync_copy(tab_hbm.at[g], stage)
        pltpu.sync_copy(z_hbm, acc)

        def win(w, carry):
            pltpu.sync_copy(src_hbm.at[pl.ds(w * _EW, _EW)], sbuf)
            pltpu.sync_copy(dst_hbm.at[pl.ds(w * _EW, _EW)], dbuf)

            def chunk(k, carry2):
                sis = [sbuf[pl.ds((k * _UNR + u) * 16, 16)]
                       for u in range(_UNR)]
                dis = [dbuf[pl.ds((k * _UNR + u) * 16, 16)]
                       for u in range(_UNR)]
                for u in range(_UNR):
                    for w2 in range(_CPW):
                        wd = plsc.load_gather(stage, [rvw[w2], sis[u]])
                        ab = plsc.bitcast(wd, jnp.bfloat16)
                        va, vb = plsc.unpack(
                            ab, format=plsc.PackFormat.INTERLEAVED)
                        plsc.addupdate_scatter(
                            acc, [rvc[2 * w2], dis[u]],
                            va.astype(jnp.float32))
                        plsc.addupdate_scatter(
                            acc, [rvc[2 * w2 + 1], dis[u]],
                            vb.astype(jnp.float32))
                return carry2
            lax.fori_loop(0, _EW // 16 // _UNR, chunk, 0)
            return carry
        lax.fori_loop(0, _NWIN, win, 0)

        pltpu.sync_copy(acc, out_hbm.at[g])


def _sc_scatter(src, dst, tab_p3, z8):
    mesh = plsc.VectorSubcoreMesh(core_axis_name="c", subcore_axis_name="s")
    return pl.kernel(
        _scat_sc_body,
        out_type=jax.ShapeDtypeStruct((_NG, _CPT, N), jnp.float32),
        mesh=mesh,
        scratch_types=[
            pltpu.VMEM((_CPW, N), jnp.int32),
            pltpu.VMEM((_CPT, N), jnp.float32),
            pltpu.VMEM((_EW,), jnp.int32),
            pltpu.VMEM((_EW,), jnp.int32),
        ],
        compiler_params=pltpu.CompilerParams(needs_layout_passes=False),
    )(src, dst, tab_p3, z8)


def kernel(x, edge_index, W1, b1, g1, beta1, Wl, bl, Wr, g2, beta2):
    src = edge_index[0].astype(jnp.int32)
    dst = edge_index[1].astype(jnp.int32)
    z8 = jnp.zeros((_CPT, N), jnp.float32)

    def pack_t3(t):
        tb = t.astype(jnp.bfloat16).reshape(N, DHID // 2, 2)
        w = jax.lax.bitcast_convert_type(tb, jnp.int32)
        return w.T.reshape(_NG, _CPW, N)

    degp = _sc_degree(dst).T
    hs, dinv, cnti = _run_k1(x, W1, degp)
    s1 = _sc_scatter(src, dst, pack_t3(hs), z8).reshape(DHID, N).T
    h2 = _run_k2(s1, hs, dinv, b1.reshape(1, DHID),
                 g1.reshape(1, DHID), beta1.reshape(1, DHID))
    s2 = _sc_scatter(src, dst, pack_t3(h2), z8).reshape(DHID, N).T
    out = _run_k3(s2, h2, cnti, Wl, Wr, bl.reshape(1, DOUT),
                  g2.reshape(1, DOUT), beta2.reshape(1, DOUT))
    return out


# speedup vs baseline: 3.4934x; 1.0003x over previous
import jax
import jax.numpy as jnp
from jax import lax
from jax.experimental import pallas as pl
from jax.experimental.pallas import tpu as pltpu
from jax.experimental.pallas import tpu_sc as plsc

N = 10000
E = 160000
DIN = 256
DHID = 512
DOUT = 256

BLK = 400
NB = N // BLK

_NC = 2
_NS = 16
_NW = _NC * _NS
_EPW = E // _NW


def _k1_body(x_ref, w_ref, degp_ref, hs_ref, dinv_ref, cnti_ref):
    h = jnp.dot(x_ref[...], w_ref[...], preferred_element_type=jnp.float32)
    deg = jnp.sum(degp_ref[...], axis=1) + 1.0
    dinv = jax.lax.rsqrt(deg)
    hs_ref[...] = h * dinv[:, None]
    dinv_ref[...] = dinv[:, None]
    cnti_ref[...] = (1.0 / jnp.maximum(deg - 1.0, 1.0))[:, None]


def _run_k1(x, W1, degp):
    return pl.pallas_call(
        _k1_body,
        grid=(NB,),
        in_specs=[
            pl.BlockSpec((BLK, DIN), lambda i: (i, 0)),
            pl.BlockSpec((DIN, DHID), lambda i: (0, 0)),
            pl.BlockSpec((BLK, _NW), lambda i: (i, 0)),
        ],
        out_specs=[
            pl.BlockSpec((BLK, DHID), lambda i: (i, 0)),
            pl.BlockSpec((BLK, 1), lambda i: (i, 0)),
            pl.BlockSpec((BLK, 1), lambda i: (i, 0)),
        ],
        out_shape=[
            jax.ShapeDtypeStruct((N, DHID), jnp.float32),
            jax.ShapeDtypeStruct((N, 1), jnp.float32),
            jax.ShapeDtypeStruct((N, 1), jnp.float32),
        ],
    )(x, W1, degp)


def _k2a_body(s1_ref, hs_ref, dinv_ref, b1_ref, pre_ref, acc_ref):
    j = pl.program_id(0)
    pre = dinv_ref[...] * (s1_ref[...] + hs_ref[...]) + b1_ref[...]
    pre_ref[...] = pre

    @pl.when(j == 0)
    def _():
        acc_ref[...] = jnp.zeros_like(acc_ref)
    acc_ref[0, :] += jnp.sum(pre, axis=0)
    acc_ref[1, :] += jnp.sum(pre * pre, axis=0)


def _bn_body(pre_ref, acc_ref, g_ref, be_ref, out_ref):
    mean = acc_ref[0, :] / N
    var = acc_ref[1, :] / N - mean * mean
    h = (pre_ref[...] - mean) * jax.lax.rsqrt(var + 1e-5) * g_ref[0, :] \
        + be_ref[0, :]
    out_ref[...] = jnp.maximum(h, 0.0)


def _run_bn(pre, acc, g, be):
    d = pre.shape[1]
    vec = pl.BlockSpec((1, d), lambda j: (0, 0))
    return pl.pallas_call(
        _bn_body,
        grid=(NB,),
        in_specs=[pl.BlockSpec((BLK, d), lambda j: (j, 0)),
                  pl.BlockSpec((2, d), lambda j: (0, 0)),
                  vec, vec],
        out_specs=pl.BlockSpec((BLK, d), lambda j: (j, 0)),
        out_shape=jax.ShapeDtypeStruct((N, d), jnp.float32),
    )(pre, acc, g, be)


def _run_k2(s1, hs, dinv, b1, g1, be1):
    row = pl.BlockSpec((BLK, DHID), lambda j: (j, 0))
    vec = pl.BlockSpec((1, DHID), lambda j: (0, 0))
    pre, acc = pl.pallas_call(
        _k2a_body,
        grid=(NB,),
        in_specs=[row, row,
                  pl.BlockSpec((BLK, 1), lambda j: (j, 0)), vec],
        out_specs=[row, pl.BlockSpec((2, DHID), lambda j: (0, 0))],
        out_shape=[
            jax.ShapeDtypeStruct((N, DHID), jnp.float32),
            jax.ShapeDtypeStruct((2, DHID), jnp.float32),
        ],
    )(s1, hs, dinv, b1)
    return _run_bn(pre, acc, g1, be1)


def _k3a_body(s2_ref, h2_ref, cnti_ref, wl_ref, wr_ref, bl_ref,
              pre_ref, acc_ref):
    j = pl.program_id(0)
    agg = s2_ref[...] * cnti_ref[...]
    pre = (jnp.dot(agg, wl_ref[...], preferred_element_type=jnp.float32)
           + jnp.dot(h2_ref[...], wr_ref[...],
                     preferred_element_type=jnp.float32)
           + bl_ref[...])
    pre_ref[...] = pre

    @pl.when(j == 0)
    def _():
        acc_ref[...] = jnp.zeros_like(acc_ref)
    acc_ref[0, :] += jnp.sum(pre, axis=0)
    acc_ref[1, :] += jnp.sum(pre * pre, axis=0)


def _run_k3(s2, h2, cnti, Wl, Wr, bl, g2, be2):
    row = pl.BlockSpec((BLK, DHID), lambda j: (j, 0))
    vec = pl.BlockSpec((1, DOUT), lambda j: (0, 0))
    wspec = pl.BlockSpec((DHID, DOUT), lambda j: (0, 0))
    pre, acc = pl.pallas_call(
        _k3a_body,
        grid=(NB,),
        in_specs=[row, row,
                  pl.BlockSpec((BLK, 1), lambda j: (j, 0)),
                  wspec, wspec, vec],
        out_specs=[pl.BlockSpec((BLK, DOUT), lambda j: (j, 0)),
                   pl.BlockSpec((2, DOUT), lambda j: (0, 0))],
        out_shape=[
            jax.ShapeDtypeStruct((N, DOUT), jnp.float32),
            jax.ShapeDtypeStruct((2, DOUT), jnp.float32),
        ],
    )(s2, h2, cnti, Wl, Wr, bl)
    return _run_bn(pre, acc, g2, be2)


def _deg_sc_body(dst_hbm, out_hbm, dst_v, hist_v):
    c = lax.axis_index("c")
    s = lax.axis_index("s")
    wid = s * _NC + c

    pltpu.sync_copy(dst_hbm.at[pl.ds(wid * _EPW, _EPW)],
                    dst_v.at[pl.ds(0, _EPW)])

    def zbody(i, carry):
        hist_v[pl.ds(i * 16, 16)] = jnp.zeros((16,), jnp.float32)
        return carry
    lax.fori_loop(0, N // 16, zbody, 0)

    ones = jnp.full((16,), 1.0, jnp.float32)
    nfull = _EPW // 16

    def body(k, carry):
        idx = dst_v[pl.ds(k * 16, 16)]
        plsc.addupdate_scatter(hist_v, [idx], ones)
        return carry
    lax.fori_loop(0, nfull, body, 0)

    rem = _EPW - nfull * 16
    if rem:
        lane = lax.broadcasted_iota(jnp.int32, (16,), 0)
        idx = dst_v[pl.ds(nfull * 16, 16)]
        plsc.addupdate_scatter(hist_v, [idx], ones, mask=lane < rem)

    pltpu.sync_copy(hist_v, out_hbm.at[wid])


def _sc_degree(dst):
    mesh = plsc.VectorSubcoreMesh(core_axis_name="c", subcore_axis_name="s")
    return pl.kernel(
        _deg_sc_body,
        out_type=jax.ShapeDtypeStruct((_NW, N), jnp.float32),
        mesh=mesh,
        scratch_types=[
            pltpu.VMEM((_EPW + 16,), jnp.int32),
            pltpu.VMEM((N,), jnp.float32),
        ],
        compiler_params=pltpu.CompilerParams(needs_layout_passes=False),
    )(dst)


_CPT = 8
_CPW = _CPT // 2
_NG = DHID // _CPT
_NP = _NG // _NW
_EW = 4000
_NWIN = E // _EW
_UNR = 10


def _scat_sc_body(src_hbm, dst_hbm, tab_hbm, z_hbm, out_hbm,
                  stage, acc, sbuf, dbuf):
    c = lax.axis_index("c")
    s = lax.axis_index("s")
    wid = s * _NC + c
    rvw = [jnp.full((16,), w, jnp.int32) for w in range(_CPW)]
    rvc = [jnp.full((16,), cc, jnp.int32) for cc in range(_CPT)]

    for p in range(_NP):
        g = p * _NW + wid
        pltpu.sync_copy(tab_hbm.at[g], stage)
        pltpu.sync_copy(z_hbm, acc)

        def win(w, carry):
            pltpu.sync_copy(src_hbm.at[pl.ds(w * _EW, _EW)], sbuf)
            pltpu.sync_copy(dst_hbm.at[pl.ds(w * _EW, _EW)], dbuf)

            def chunk(k, carry2):
                sis = [sbuf[pl.ds((k * _UNR + u) * 16, 16)]
                       for u in range(_UNR)]
                dis = [dbuf[pl.ds((k * _UNR + u) * 16, 16)]
                       for u in range(_UNR)]
                for u in range(_UNR):
                    for w2 in range(_CPW):
                        wd = plsc.load_gather(stage, [rvw[w2], sis[u]])
                        ab = plsc.bitcast(wd, jnp.bfloat16)
                        va, vb = plsc.unpack(
                            ab, format=plsc.PackFormat.INTERLEAVED)
                        plsc.addupdate_scatter(
                            acc, [rvc[2 * w2], dis[u]],
                            va.astype(jnp.float32))
                        plsc.addupdate_scatter(
                            acc, [rvc[2 * w2 + 1], dis[u]],
                            vb.astype(jnp.float32))
                return carry2
            lax.fori_loop(0, _EW // 16 // _UNR, chunk, 0)
            return carry
        lax.fori_loop(0, _NWIN, win, 0)

        pltpu.sync_copy(acc, out_hbm.at[g])


def _sc_scatter(src, dst, tab_p3, z8):
    mesh = plsc.VectorSubcoreMesh(core_axis_name="c", subcore_axis_name="s")
    return pl.kernel(
        _scat_sc_body,
        out_type=jax.ShapeDtypeStruct((_NG, _CPT, N), jnp.float32),
        mesh=mesh,
        scratch_types=[
            pltpu.VMEM((_CPW, N), jnp.int32),
            pltpu.VMEM((_CPT, N), jnp.float32),
            pltpu.VMEM((_EW,), jnp.int32),
            pltpu.VMEM((_EW,), jnp.int32),
        ],
        compiler_params=pltpu.CompilerParams(needs_layout_passes=False),
    )(src, dst, tab_p3, z8)


def kernel(x, edge_index, W1, b1, g1, beta1, Wl, bl, Wr, g2, beta2):
    src = edge_index[0].astype(jnp.int32)
    dst = edge_index[1].astype(jnp.int32)
    z8 = jnp.zeros((_CPT, N), jnp.float32)

    def pack_t3(t):
        tb = t.astype(jnp.bfloat16).reshape(N, DHID // 2, 2)
        w = jax.lax.bitcast_convert_type(tb, jnp.int32)
        return w.T.reshape(_NG, _CPW, N)

    degp = _sc_degree(dst).T
    hs, dinv, cnti = _run_k1(x, W1, degp)
    s1 = _sc_scatter(src, dst, pack_t3(hs), z8).reshape(DHID, N).T
    h2 = _run_k2(s1, hs, dinv, b1.reshape(1, DHID),
                 g1.reshape(1, DHID), beta1.reshape(1, DHID))
    s2 = _sc_scatter(src, dst, pack_t3(h2), z8).reshape(DHID, N).T
    out = _run_k3(s2, h2, cnti, Wl, Wr, bl.reshape(1, DOUT),
                  g2.reshape(1, DOUT), beta2.reshape(1, DOUT))
    return out
